# baseline (speedup 1.0000x reference)
import functools

import jax
import jax.numpy as jnp
from jax import lax
from jax.experimental import pallas as pl
from jax.experimental.pallas import tpu as pltpu
from jax.experimental.pallas import tpu_sc as plsc

VOCAB = 1000000
EMBED = 64

NC = 2
NS = 16
NW = NC * NS

CHUNK = 128
B_TOTAL = 16384 * 50
N_IDX_ROWS = B_TOTAL // CHUNK
ROWS_PER_W = N_IDX_ROWS // NW
NBUF = 8

N_SLAB_FULL = VOCAB // CHUNK
TAIL = VOCAB - N_SLAB_FULL * CHUNK
SLAB_MAIN = (N_SLAB_FULL // NW) * NW
TNBUF = 4


def _mesh():
    return plsc.VectorSubcoreMesh(
        core_axis_name="c", subcore_axis_name="s", num_cores=NC,
        num_subcores=NS)


PITCH = EMBED + 1


def _transpose_slab(src, width, b2p):
    lanes = lax.iota(jnp.int32, 16) * PITCH

    def body(e):
        for j0 in range(width // 16):
            x = src[e, pl.ds(j0 * 16, 16)]
            plsc.store_scatter(b2p, [lanes + (j0 * 16 * PITCH + e)], x)

    plsc.parallel_loop(0, EMBED, unroll=4)(body)


def _sc_transpose_table(table_t):
    @functools.partial(
        pl.kernel,
        out_type=jax.ShapeDtypeStruct((VOCAB * PITCH,), jnp.float32),
        mesh=_mesh(),
        compiler_params=pltpu.CompilerParams(needs_layout_passes=False),
        scratch_types=(
            [pltpu.VMEM((EMBED, CHUNK), jnp.float32)] * TNBUF
            + [pltpu.VMEM((CHUNK * PITCH,), jnp.float32)] * TNBUF
            + [
                pltpu.VMEM((EMBED, TAIL), jnp.float32),
                pltpu.VMEM((TAIL * PITCH,), jnp.float32),
                pltpu.SemaphoreType.DMA,
                pltpu.SemaphoreType.DMA,
            ]
        ),
    )
    def k(tab_hbm, out_hbm, *scratch):
        s_v = scratch[:TNBUF]
        b2_v = scratch[TNBUF:2 * TNBUF]
        st_v, bt_v, isem, osem = scratch[2 * TNBUF:]
        wid = lax.axis_index("s") * NC + lax.axis_index("c")

        def col0(i):
            return pl.multiple_of((wid + NW * i) * CHUNK, CHUNK)

        def start_in(i, b):
            pltpu.async_copy(
                tab_hbm.at[:, pl.ds(col0(i), CHUNK)], s_v[b], isem)

        def wait_in(b):
            pltpu.make_async_copy(
                tab_hbm.at[:, pl.ds(0, CHUNK)], s_v[b], isem).wait()

        def start_out(i, b):
            pltpu.async_copy(
                b2_v[b],
                out_hbm.at[pl.ds(col0(i) * PITCH, CHUNK * PITCH)], osem)

        def wait_out(b):
            pltpu.make_async_copy(
                b2_v[0],
                out_hbm.at[pl.ds(0, CHUNK * PITCH)], osem).wait()

        n_main = SLAB_MAIN // NW
        for b in range(TNBUF):
            start_in(b, b)

        def body(g):
            for b in range(TNBUF):
                i = g + b
                wait_in(b)

                @pl.when(i >= TNBUF)
                def _():
                    wait_out(b)

                _transpose_slab(s_v[b], CHUNK, b2_v[b])
                start_out(i, b)

                @pl.when(i + TNBUF < n_main)
                def _():
                    start_in(i + TNBUF, b)

        pl.loop(0, n_main, step=TNBUF)(body)
        for b in range(TNBUF):
            wait_out(b)

        @pl.when(wid < N_SLAB_FULL - SLAB_MAIN)
        def _():
            c0 = pl.multiple_of((SLAB_MAIN + wid) * CHUNK, CHUNK)
            pltpu.sync_copy(tab_hbm.at[:, pl.ds(c0, CHUNK)], s_v[0])
            _transpose_slab(s_v[0], CHUNK, b2_v[0])
            pltpu.sync_copy(
                b2_v[0], out_hbm.at[pl.ds(c0 * PITCH, CHUNK * PITCH)])

        @pl.when(wid == NW - 1)
        def _():
            t0 = N_SLAB_FULL * CHUNK
            pltpu.sync_copy(tab_hbm.at[:, pl.ds(t0, TAIL)], st_v)
            _transpose_slab(st_v, TAIL, bt_v)
            pltpu.sync_copy(
                bt_v, out_hbm.at[pl.ds(t0 * PITCH, TAIL * PITCH)])

    return k(table_t)


def _sc_gather(idx2d, table):
    @functools.partial(
        pl.kernel,
        out_type=jax.ShapeDtypeStruct((B_TOTAL, EMBED), jnp.float32),
        mesh=_mesh(),
        compiler_params=pltpu.CompilerParams(use_tc_tiling_on_sc=False),
        scratch_types=[
            pltpu.VMEM((ROWS_PER_W, CHUNK), jnp.int32),
            pltpu.VMEM((NBUF, CHUNK, PITCH), jnp.float32),
            pltpu.SemaphoreType.DMA,
            pltpu.SemaphoreType.DMA,
        ],
    )
    def k(idx_hbm, table_hbm, out_hbm, idx_v, rows_v, gsem, ssem):
        wid = lax.axis_index("s") * NC + lax.axis_index("c")
        idx_base = wid * ROWS_PER_W
        pltpu.sync_copy(idx_hbm.at[pl.ds(idx_base, ROWS_PER_W)], idx_v)

        def gather(j, b):
            pltpu.async_copy(table_hbm.at[idx_v.at[j]], rows_v.at[b], gsem)

        def store(j, b):
            pltpu.async_copy(
                rows_v.at[b, :, pl.ds(0, EMBED)],
                out_hbm.at[pl.ds((idx_base + j) * CHUNK, CHUNK)], ssem)

        def wait_gather(b):
            pltpu.make_async_copy(
                table_hbm.at[idx_v.at[0]], rows_v.at[b], gsem).wait()

        def wait_store(b):
            pltpu.make_async_copy(
                rows_v.at[b, :, pl.ds(0, EMBED)],
                out_hbm.at[pl.ds(0, CHUNK)], ssem).wait()

        for b in range(NBUF):
            gather(b, b)
        for b in range(NBUF):
            wait_gather(b)
            store(b, b)

        def body(g):
            for b in range(NBUF):
                wait_store(b)
                gather(g + b, b)
            for b in range(NBUF):
                wait_gather(b)
                store(g + b, b)

        pl.loop(NBUF, ROWS_PER_W, step=NBUF)(body)

        for b in range(NBUF):
            wait_store(b)

    return k(idx2d, table)


def kernel(inputs, embeddings):
    table_lin = _sc_transpose_table(embeddings.T)
    idx2d = inputs.reshape(N_IDX_ROWS, CHUNK).astype(jnp.int32)
    out = _sc_gather(idx2d, table_lin.reshape(VOCAB, PITCH))
    return out.reshape(inputs.shape[0], inputs.shape[1], EMBED)


# speedup vs baseline: 1.9482x; 1.9482x over previous
import functools

import jax
import jax.numpy as jnp
from jax import lax
from jax.experimental import pallas as pl
from jax.experimental.pallas import tpu as pltpu
from jax.experimental.pallas import tpu_sc as plsc

VOCAB = 1000000
EMBED = 64

NC = 2
NS = 16
NW = NC * NS

CHUNK = 128
B_TOTAL = 16384 * 50
N_IDX_ROWS = B_TOTAL // CHUNK
ROWS_PER_W = N_IDX_ROWS // NW
NBUF = 8


def _sc_gather(idx2d, table):
    mesh = plsc.VectorSubcoreMesh(
        core_axis_name="c", subcore_axis_name="s", num_cores=NC,
        num_subcores=NS)

    @functools.partial(
        pl.kernel,
        out_type=jax.ShapeDtypeStruct((B_TOTAL, EMBED), jnp.float32),
        mesh=mesh,
        compiler_params=pltpu.CompilerParams(use_tc_tiling_on_sc=False),
        scratch_types=[
            pltpu.VMEM((ROWS_PER_W, CHUNK), jnp.int32),
            pltpu.VMEM((NBUF, CHUNK, EMBED), jnp.float32),
            pltpu.SemaphoreType.DMA,
            pltpu.SemaphoreType.DMA,
        ],
    )
    def k(idx_hbm, table_hbm, out_hbm, idx_v, rows_v, gsem, ssem):
        wid = lax.axis_index("s") * NC + lax.axis_index("c")
        idx_base = wid * ROWS_PER_W
        pltpu.sync_copy(idx_hbm.at[pl.ds(idx_base, ROWS_PER_W)], idx_v)

        def gather(j, b):
            pltpu.async_copy(table_hbm.at[idx_v.at[j]], rows_v.at[b], gsem)

        def store(j, b):
            pltpu.async_copy(
                rows_v.at[b], out_hbm.at[pl.ds((idx_base + j) * CHUNK, CHUNK)],
                ssem)

        def wait_gather(b):
            pltpu.make_async_copy(
                table_hbm.at[idx_v.at[0]], rows_v.at[b], gsem).wait()

        def wait_store(b):
            pltpu.make_async_copy(
                rows_v.at[b], out_hbm.at[pl.ds(0, CHUNK)], ssem).wait()

        for b in range(NBUF):
            gather(b, b)
        for b in range(NBUF):
            wait_gather(b)
            store(b, b)

        def body(g):
            for b in range(NBUF):
                wait_store(b)
                gather(g + b, b)
            for b in range(NBUF):
                wait_gather(b)
                store(g + b, b)

        pl.loop(NBUF, ROWS_PER_W, step=NBUF)(body)

        for b in range(NBUF):
            wait_store(b)

    return k(idx2d, table)


def kernel(inputs, embeddings):
    idx2d = inputs.reshape(N_IDX_ROWS, CHUNK).astype(jnp.int32)
    out = _sc_gather(idx2d, embeddings)
    return out.reshape(inputs.shape[0], inputs.shape[1], EMBED)
